# Initial kernel scaffold; baseline (speedup 1.0000x reference)
import jax
import jax.numpy as jnp
from jax.experimental import pallas as pl


def kernel(edge_index, edge_type, emb, proj_W, proj_b, basis0, comp0, root0, bias0, g0, b0, basis1, comp1, root1, bias1, g1, b1, basis2, comp2, root2, bias2, g2, b2):
    raise NotImplementedError("write your pallas kernel here")


# speedup vs baseline: 10.1261x; 10.1261x over previous
import functools

import jax
import jax.numpy as jnp
from jax import lax
from jax.experimental import pallas as pl
from jax.experimental.pallas import tpu as pltpu
from jax.experimental.pallas import tpu_sc as plsc

N = 10000
E = 320000
R = 8
NB = 2
P = 768
H = 128

NC = 2
NS = 16
NW = NC * NS
CHUNK = 128
EPAD = 327680
ROWS = EPAD // CHUNK
TROWS = ROWS // NW
CROWS = ROWS // NS
NR = N * R
NRP = 81920
FR = 624
ZR = 48
TBAT = 16

def _mesh():
    return plsc.VectorSubcoreMesh(core_axis_name="c", subcore_axis_name="s",
                                  num_cores=NC, num_subcores=NS)


def _zero16():
    return jnp.zeros((16,), jnp.float32)


@functools.cache
def _build_counts_weights():
  return functools.partial(
    pl.kernel,
    out_type=jax.ShapeDtypeStruct((ROWS, CHUNK), jnp.float32),
    mesh=_mesh(),
    scratch_types=[
        pltpu.VMEM((CROWS, CHUNK), jnp.int32),
        pltpu.VMEM((TROWS, CHUNK), jnp.int32),
        pltpu.VMEM((TROWS, CHUNK), jnp.float32),
        pltpu.VMEM((CHUNK,), jnp.float32),
        pltpu.VMEM((CHUNK,), jnp.float32),
        pltpu.VMEM_SHARED((NRP,), jnp.float32),
        pltpu.SemaphoreType.DMA,
        pltpu.SemaphoreType.DMA,
    ],
  )(_counts_weights_body)


def _sc_counts_weights(seg):
    return _build_counts_weights()(seg)


def _counts_weights_body(seg_hbm, w_hbm, segc_v, segw_v, cw_v, ones_v, zeros_v,
                         cnt_sh, sem, sem2):
    cid = lax.axis_index("c")
    sid = lax.axis_index("s")
    wid = sid * NC + cid

    for k in range(CHUNK // 16):
        ones_v[pl.ds(k * 16, 16)] = jnp.full((16,), 1.0, jnp.float32)
        zeros_v[pl.ds(k * 16, 16)] = _zero16()
    for t in range(NRP // NS // CHUNK):
        pltpu.sync_copy(zeros_v, cnt_sh.at[pl.ds(sid * (NRP // NS) + t * CHUNK, CHUNK)])
    plsc.subcore_barrier()

    pltpu.sync_copy(seg_hbm.at[pl.ds(sid * CROWS, CROWS)], segc_v)
    copies = []
    for t in range(CROWS):
        copies.append(pltpu.make_async_copy(ones_v, cnt_sh.at[segc_v.at[t]], sem))
        copies[-1].start(add=True)
    for c in copies:
        c.wait()
    plsc.subcore_barrier()

    pltpu.sync_copy(seg_hbm.at[pl.ds(wid * TROWS, TROWS)], segw_v)
    gathers = []
    for t in range(TROWS):
        gathers.append(pltpu.make_async_copy(cnt_sh.at[segw_v.at[t]],
                                             cw_v.at[t], sem2))
        gathers[-1].start()
    for g in gathers:
        g.wait()

    base = wid * TROWS * CHUNK

    def body(i, _):
        j = i // (CHUNK // 16)
        k = i % (CHUNK // 16)
        c = cw_v[j, pl.ds(k * 16, 16)]
        w = 1.0 / jnp.maximum(c, 1.0)
        gidx = base + i * 16 + lax.broadcasted_iota(jnp.int32, (16,), 0)
        cw_v[j, pl.ds(k * 16, 16)] = jnp.where(gidx < E, w, 0.0)
        return 0

    lax.fori_loop(0, TROWS * (CHUNK // 16), body, 0)
    pltpu.sync_copy(cw_v, w_hbm.at[pl.ds(wid * TROWS, TROWS)])


@functools.cache
def _build_scatter():
  return functools.partial(
    pl.kernel,
    out_type=jax.ShapeDtypeStruct((NC, N, H), jnp.float32),
    mesh=_mesh(),
    scratch_types=[
        pltpu.VMEM((TBAT, CHUNK), jnp.int32),
        pltpu.VMEM((TBAT, CHUNK), jnp.int32),
        pltpu.VMEM((TBAT * CHUNK,), jnp.float32),
        pltpu.VMEM((CHUNK, H), jnp.float32),
        pltpu.VMEM((ZR, H), jnp.float32),
        pltpu.VMEM_SHARED((N, H), jnp.float32),
        pltpu.SemaphoreType.DMA,
    ],
  )(_scatter_body)


def _sc_scatter(srel, dstp, w, xr):
    return _build_scatter()(srel, dstp, w, xr)


def _scatter_body(srel_hbm, dst_hbm, w_hbm, xr_hbm, out_hbm,
                  srel_v, dst_v, w_v, rows_v, zero_v, agg_sh, sem):
    cid = lax.axis_index("c")
    sid = lax.axis_index("s")
    wid = sid * NC + cid

    def zbody(i, _):
        j = i // (H // 16)
        k = i % (H // 16)
        zero_v[j, pl.ds(k * 16, 16)] = _zero16()
        return 0

    lax.fori_loop(0, ZR * (H // 16), zbody, 0)
    for t in range(FR // ZR):
        pltpu.sync_copy(zero_v, agg_sh.at[pl.ds(sid * FR + t * ZR, ZR)])

    @pl.when(sid == NS - 1)
    def _():
        pltpu.sync_copy(zero_v.at[pl.ds(0, N - FR * NS)],
                        agg_sh.at[pl.ds(FR * NS, N - FR * NS)])

    plsc.subcore_barrier()

    for bt in range(TROWS // TBAT):
        base = wid * TROWS + bt * TBAT
        pltpu.sync_copy(srel_hbm.at[pl.ds(base, TBAT)], srel_v)
        pltpu.sync_copy(dst_hbm.at[pl.ds(base, TBAT)], dst_v)
        pltpu.sync_copy(w_hbm.at[pl.ds(base * CHUNK, TBAT * CHUNK)], w_v)

        def tbody(t, _):
            pltpu.async_copy(xr_hbm.at[srel_v.at[t]], rows_v, sem).wait()

            def sbody(g, _):
                w16 = w_v[pl.ds(t * CHUNK + g * 16, 16)]
                for j in range(16):
                    wb = jnp.full((16,), w16[j], jnp.float32)
                    e = g * 16 + j
                    for k in range(H // 16):
                        rows_v[e, pl.ds(k * 16, 16)] = (
                            rows_v[e, pl.ds(k * 16, 16)] * wb)
                return 0

            lax.fori_loop(0, CHUNK // 16, sbody, 0)
            pltpu.sync_copy(rows_v, agg_sh.at[dst_v.at[t]], add=True)
            return 0

        lax.fori_loop(0, TBAT, tbody, 0)

    plsc.subcore_barrier()
    for t in range(FR // ZR):
        pltpu.sync_copy(agg_sh.at[pl.ds(sid * FR + t * ZR, ZR)], zero_v)
        pltpu.sync_copy(zero_v, out_hbm.at[cid, pl.ds(sid * FR + t * ZR, ZR)])

    @pl.when(sid == NS - 1)
    def _():
        pltpu.sync_copy(agg_sh.at[pl.ds(FR * NS, N - FR * NS)],
                        zero_v.at[pl.ds(0, N - FR * NS)])
        pltpu.sync_copy(zero_v.at[pl.ds(0, N - FR * NS)],
                        out_hbm.at[cid, pl.ds(FR * NS, N - FR * NS)])


def _tc_proj_kernel(emb_ref, w_ref, b_ref, out_ref):
    x = emb_ref[...]
    nrm = jnp.sqrt(jnp.sum(x * x, axis=1, keepdims=True))
    x = x / jnp.maximum(nrm, 1e-12)
    out_ref[...] = jnp.dot(x, w_ref[...],
                           preferred_element_type=jnp.float32) + b_ref[...]


def _tc_proj(emb, proj_W, proj_b):
    blk = 2000
    return pl.pallas_call(
        _tc_proj_kernel,
        grid=(N // blk,),
        in_specs=[
            pl.BlockSpec((blk, emb.shape[1]), lambda i: (i, 0)),
            pl.BlockSpec(proj_W.shape, lambda i: (0, 0)),
            pl.BlockSpec((1, proj_W.shape[1]), lambda i: (0, 0)),
        ],
        out_specs=pl.BlockSpec((blk, proj_W.shape[1]), lambda i: (i, 0)),
        out_shape=jax.ShapeDtypeStruct((N, proj_W.shape[1]), jnp.float32),
    )(emb, proj_W, proj_b.reshape(1, -1))


def _tc_rel_matmul_kernel(x_ref, basis_ref, comp_ref, root_ref,
                          xr_ref, xroot_ref, wcat_ref):
    @pl.when(pl.program_id(0) == 0)
    def _():
        b0 = basis_ref[0]
        b1 = basis_ref[1]
        for r in range(R):
            wcat_ref[:, r * H:(r + 1) * H] = comp_ref[r, 0] * b0 + comp_ref[r, 1] * b1
        wcat_ref[:, R * H:] = root_ref[...]

    y = jnp.dot(x_ref[...], wcat_ref[...], preferred_element_type=jnp.float32)
    xr_ref[...] = y[:, :R * H]
    xroot_ref[...] = y[:, R * H:]


def _tc_rel_matmul(x, basis, comp, root):
    din = x.shape[1]
    blk = 2000
    comp_p = jnp.zeros((R, 128), jnp.float32).at[:, :NB].set(comp)
    return pl.pallas_call(
        _tc_rel_matmul_kernel,
        grid=(N // blk,),
        in_specs=[
            pl.BlockSpec((blk, din), lambda i: (i, 0)),
            pl.BlockSpec((NB, din, H), lambda i: (0, 0, 0)),
            pl.BlockSpec((R, 128), lambda i: (0, 0)),
            pl.BlockSpec((din, H), lambda i: (0, 0)),
        ],
        out_specs=[
            pl.BlockSpec((blk, R * H), lambda i: (i, 0)),
            pl.BlockSpec((blk, H), lambda i: (i, 0)),
        ],
        out_shape=[
            jax.ShapeDtypeStruct((N, R * H), jnp.float32),
            jax.ShapeDtypeStruct((N, H), jnp.float32),
        ],
        scratch_shapes=[pltpu.VMEM((din, R * H + H), jnp.float32)],
    )(x, basis, comp_p, root)


def _tc_combine_kernel(agg_ref, xroot_ref, g_ref, b_ref, out_ref):
    s = agg_ref[0] + agg_ref[1] + xroot_ref[...]
    m = jnp.sum(s, axis=0, keepdims=True) / N
    v = jnp.sum(s * s, axis=0, keepdims=True) / N - m * m
    y = (s - m) * jax.lax.rsqrt(v + 1e-5) * g_ref[...] + b_ref[...]
    out_ref[...] = jnp.maximum(y, 0.0)


def _tc_combine(agg2, xroot, g, b):
    return pl.pallas_call(
        _tc_combine_kernel,
        in_specs=[
            pl.BlockSpec((NC, N, H), lambda: (0, 0, 0)),
            pl.BlockSpec((N, H), lambda: (0, 0)),
            pl.BlockSpec((1, H), lambda: (0, 0)),
            pl.BlockSpec((1, H), lambda: (0, 0)),
        ],
        out_specs=pl.BlockSpec((N, H), lambda: (0, 0)),
        out_shape=jax.ShapeDtypeStruct((N, H), jnp.float32),
    )(agg2, xroot, g.reshape(1, -1), b.reshape(1, -1))


def kernel(edge_index, edge_type, emb, proj_W, proj_b,
           basis0, comp0, root0, bias0, g0, b0,
           basis1, comp1, root1, bias1, g1, b1,
           basis2, comp2, root2, bias2, g2, b2):
    src = edge_index[0].astype(jnp.int32)
    dst = edge_index[1].astype(jnp.int32)
    et = edge_type.astype(jnp.int32)

    pad = EPAD - E
    srel = jnp.concatenate([src * R + et, jnp.zeros((pad,), jnp.int32)])
    seg = jnp.concatenate([dst * R + et, jnp.full((pad,), NR, jnp.int32)])
    dstp = jnp.concatenate([dst, jnp.zeros((pad,), jnp.int32)])
    srel = srel.reshape(ROWS, CHUNK)
    seg = seg.reshape(ROWS, CHUNK)
    dstp = dstp.reshape(ROWS, CHUNK)

    w = _sc_counts_weights(seg)

    x = _tc_proj(emb, proj_W, proj_b)
    for basis, comp, root, g, b in (
            (basis0, comp0, root0, g0, b0),
            (basis1, comp1, root1, g1, b1),
            (basis2, comp2, root2, g2, b2)):
        xr, xroot = _tc_rel_matmul(x, basis, comp, root)
        agg2 = _sc_scatter(srel, dstp, w.reshape(EPAD), xr.reshape(NR, H))
        x = _tc_combine(agg2, xroot, g, b)
    return x


# speedup vs baseline: 11.5091x; 1.1366x over previous
import functools

import jax
import jax.numpy as jnp
from jax import lax
from jax.experimental import pallas as pl
from jax.experimental.pallas import tpu as pltpu
from jax.experimental.pallas import tpu_sc as plsc

N = 10000
E = 320000
R = 8
NB = 2
P = 768
H = 128

NC = 2
NS = 16
NW = NC * NS
CHUNK = 128
EPAD = 327680
ROWS = EPAD // CHUNK
TROWS = ROWS // NW
CROWS = ROWS // NS
NR = N * R
NRP = 81920
FR = 624
ZR = 48
TBAT = 16

def _mesh():
    return plsc.VectorSubcoreMesh(core_axis_name="c", subcore_axis_name="s",
                                  num_cores=NC, num_subcores=NS)


def _zero16():
    return jnp.zeros((16,), jnp.float32)


@functools.cache
def _build_counts_weights():
  return functools.partial(
    pl.kernel,
    out_type=jax.ShapeDtypeStruct((ROWS, CHUNK), jnp.float32),
    mesh=_mesh(),
    scratch_types=[
        pltpu.VMEM((CROWS, CHUNK), jnp.int32),
        pltpu.VMEM((TROWS, CHUNK), jnp.int32),
        pltpu.VMEM((TROWS, CHUNK), jnp.float32),
        pltpu.VMEM((CHUNK,), jnp.float32),
        pltpu.VMEM((CHUNK,), jnp.float32),
        pltpu.VMEM_SHARED((NRP,), jnp.float32),
        pltpu.SemaphoreType.DMA,
        pltpu.SemaphoreType.DMA,
    ],
  )(_counts_weights_body)


def _sc_counts_weights(seg):
    return _build_counts_weights()(seg)


def _counts_weights_body(seg_hbm, w_hbm, segc_v, segw_v, cw_v, ones_v, zeros_v,
                         cnt_sh, sem, sem2):
    cid = lax.axis_index("c")
    sid = lax.axis_index("s")
    wid = sid * NC + cid

    for k in range(CHUNK // 16):
        ones_v[pl.ds(k * 16, 16)] = jnp.full((16,), 1.0, jnp.float32)
        zeros_v[pl.ds(k * 16, 16)] = _zero16()
    for t in range(NRP // NS // CHUNK):
        pltpu.sync_copy(zeros_v, cnt_sh.at[pl.ds(sid * (NRP // NS) + t * CHUNK, CHUNK)])
    plsc.subcore_barrier()

    pltpu.sync_copy(seg_hbm.at[pl.ds(sid * CROWS, CROWS)], segc_v)
    copies = []
    for t in range(CROWS):
        copies.append(pltpu.make_async_copy(ones_v, cnt_sh.at[segc_v.at[t]], sem))
        copies[-1].start(add=True)
    for c in copies:
        c.wait()
    plsc.subcore_barrier()

    pltpu.sync_copy(seg_hbm.at[pl.ds(wid * TROWS, TROWS)], segw_v)
    gathers = []
    for t in range(TROWS):
        gathers.append(pltpu.make_async_copy(cnt_sh.at[segw_v.at[t]],
                                             cw_v.at[t], sem2))
        gathers[-1].start()
    for g in gathers:
        g.wait()

    base = wid * TROWS * CHUNK

    def body(i, _):
        j = i // (CHUNK // 16)
        k = i % (CHUNK // 16)
        c = cw_v[j, pl.ds(k * 16, 16)]
        w = 1.0 / jnp.maximum(c, 1.0)
        gidx = base + i * 16 + lax.broadcasted_iota(jnp.int32, (16,), 0)
        cw_v[j, pl.ds(k * 16, 16)] = jnp.where(gidx < E, w, 0.0)
        return 0

    lax.fori_loop(0, TROWS * (CHUNK // 16), body, 0)
    pltpu.sync_copy(cw_v, w_hbm.at[pl.ds(wid * TROWS, TROWS)])


@functools.cache
def _build_scatter():
  return functools.partial(
    pl.kernel,
    out_type=jax.ShapeDtypeStruct((NC, N, H), jnp.float32),
    mesh=_mesh(),
    scratch_types=[
        pltpu.VMEM((TBAT, CHUNK), jnp.int32),
        pltpu.VMEM((TBAT, CHUNK), jnp.int32),
        pltpu.VMEM((TBAT * CHUNK,), jnp.float32),
        pltpu.VMEM((CHUNK, H), jnp.float32),
        pltpu.VMEM((CHUNK, H), jnp.float32),
        pltpu.VMEM((ZR, H), jnp.float32),
        pltpu.VMEM_SHARED((N, H), jnp.float32),
        pltpu.SemaphoreType.DMA,
        pltpu.SemaphoreType.DMA,
    ],
  )(_scatter_body)


def _sc_scatter(srel, dstp, w, xr):
    return _build_scatter()(srel, dstp, w, xr)


def _scatter_body(srel_hbm, dst_hbm, w_hbm, xr_hbm, out_hbm,
                  srel_v, dst_v, w_v, rows0_v, rows1_v, zero_v, agg_sh,
                  semg, sems):
    cid = lax.axis_index("c")
    sid = lax.axis_index("s")
    wid = sid * NC + cid

    def zbody(i, _):
        j = i // (H // 16)
        k = i % (H // 16)
        zero_v[j, pl.ds(k * 16, 16)] = _zero16()
        return 0

    lax.fori_loop(0, ZR * (H // 16), zbody, 0)
    for t in range(FR // ZR):
        pltpu.sync_copy(zero_v, agg_sh.at[pl.ds(sid * FR + t * ZR, ZR)])

    @pl.when(sid == NS - 1)
    def _():
        pltpu.sync_copy(zero_v.at[pl.ds(0, N - FR * NS)],
                        agg_sh.at[pl.ds(FR * NS, N - FR * NS)])

    plsc.subcore_barrier()

    def scale(rows_v, t):
        def sbody(g, _):
            w16 = w_v[pl.ds(t * CHUNK + g * 16, 16)]
            for j in range(16):
                wb = jnp.full((16,), w16[j], jnp.float32)
                e = g * 16 + j
                for k in range(H // 16):
                    rows_v[e, pl.ds(k * 16, 16)] = (
                        rows_v[e, pl.ds(k * 16, 16)] * wb)
            return 0

        lax.fori_loop(0, CHUNK // 16, sbody, 0)

    for bt in range(TROWS // TBAT):
        base = wid * TROWS + bt * TBAT
        pltpu.sync_copy(srel_hbm.at[pl.ds(base, TBAT)], srel_v)
        pltpu.sync_copy(dst_hbm.at[pl.ds(base, TBAT)], dst_v)
        pltpu.sync_copy(w_hbm.at[pl.ds(base * CHUNK, TBAT * CHUNK)], w_v)

        pltpu.async_copy(xr_hbm.at[srel_v.at[0]], rows0_v, semg)

        def pbody(i, _):
            t0 = 2 * i
            t1 = t0 + 1
            pltpu.make_async_copy(xr_hbm.at[srel_v.at[t0]], rows0_v,
                                  semg).wait()

            @pl.when(i > 0)
            def _():
                pltpu.make_async_copy(rows1_v, agg_sh.at[dst_v.at[t0 - 1]],
                                      sems).wait()

            pltpu.async_copy(xr_hbm.at[srel_v.at[t1]], rows1_v, semg)
            scale(rows0_v, t0)
            pltpu.async_copy(rows0_v, agg_sh.at[dst_v.at[t0]], sems, add=True)

            pltpu.make_async_copy(xr_hbm.at[srel_v.at[t1]], rows1_v,
                                  semg).wait()
            pltpu.make_async_copy(rows0_v, agg_sh.at[dst_v.at[t0]],
                                  sems).wait()

            @pl.when(i < TBAT // 2 - 1)
            def _():
                pltpu.async_copy(xr_hbm.at[srel_v.at[t0 + 2]], rows0_v, semg)

            scale(rows1_v, t1)
            pltpu.async_copy(rows1_v, agg_sh.at[dst_v.at[t1]], sems, add=True)
            return 0

        lax.fori_loop(0, TBAT // 2, pbody, 0)
        pltpu.make_async_copy(rows1_v, agg_sh.at[dst_v.at[TBAT - 1]],
                              sems).wait()

    plsc.subcore_barrier()
    for t in range(FR // ZR):
        pltpu.sync_copy(agg_sh.at[pl.ds(sid * FR + t * ZR, ZR)], zero_v)
        pltpu.sync_copy(zero_v, out_hbm.at[cid, pl.ds(sid * FR + t * ZR, ZR)])

    @pl.when(sid == NS - 1)
    def _():
        pltpu.sync_copy(agg_sh.at[pl.ds(FR * NS, N - FR * NS)],
                        zero_v.at[pl.ds(0, N - FR * NS)])
        pltpu.sync_copy(zero_v.at[pl.ds(0, N - FR * NS)],
                        out_hbm.at[cid, pl.ds(FR * NS, N - FR * NS)])


def _tc_proj_kernel(emb_ref, w_ref, b_ref, out_ref):
    x = emb_ref[...]
    nrm = jnp.sqrt(jnp.sum(x * x, axis=1, keepdims=True))
    x = x / jnp.maximum(nrm, 1e-12)
    out_ref[...] = jnp.dot(x, w_ref[...],
                           preferred_element_type=jnp.float32) + b_ref[...]


def _tc_proj(emb, proj_W, proj_b):
    blk = 2000
    return pl.pallas_call(
        _tc_proj_kernel,
        grid=(N // blk,),
        in_specs=[
            pl.BlockSpec((blk, emb.shape[1]), lambda i: (i, 0)),
            pl.BlockSpec(proj_W.shape, lambda i: (0, 0)),
            pl.BlockSpec((1, proj_W.shape[1]), lambda i: (0, 0)),
        ],
        out_specs=pl.BlockSpec((blk, proj_W.shape[1]), lambda i: (i, 0)),
        out_shape=jax.ShapeDtypeStruct((N, proj_W.shape[1]), jnp.float32),
    )(emb, proj_W, proj_b.reshape(1, -1))


def _tc_rel_matmul_kernel(x_ref, basis_ref, comp_ref, root_ref,
                          xr_ref, xroot_ref, wcat_ref):
    @pl.when(pl.program_id(0) == 0)
    def _():
        b0 = basis_ref[0]
        b1 = basis_ref[1]
        for r in range(R):
            wcat_ref[:, r * H:(r + 1) * H] = comp_ref[r, 0] * b0 + comp_ref[r, 1] * b1
        wcat_ref[:, R * H:] = root_ref[...]

    y = jnp.dot(x_ref[...], wcat_ref[...], preferred_element_type=jnp.float32)
    xr_ref[...] = y[:, :R * H]
    xroot_ref[...] = y[:, R * H:]


def _tc_rel_matmul(x, basis, comp, root):
    din = x.shape[1]
    blk = 2000
    comp_p = jnp.zeros((R, 128), jnp.float32).at[:, :NB].set(comp)
    return pl.pallas_call(
        _tc_rel_matmul_kernel,
        grid=(N // blk,),
        in_specs=[
            pl.BlockSpec((blk, din), lambda i: (i, 0)),
            pl.BlockSpec((NB, din, H), lambda i: (0, 0, 0)),
            pl.BlockSpec((R, 128), lambda i: (0, 0)),
            pl.BlockSpec((din, H), lambda i: (0, 0)),
        ],
        out_specs=[
            pl.BlockSpec((blk, R * H), lambda i: (i, 0)),
            pl.BlockSpec((blk, H), lambda i: (i, 0)),
        ],
        out_shape=[
            jax.ShapeDtypeStruct((N, R * H), jnp.float32),
            jax.ShapeDtypeStruct((N, H), jnp.float32),
        ],
        scratch_shapes=[pltpu.VMEM((din, R * H + H), jnp.float32)],
    )(x, basis, comp_p, root)


def _tc_combine_kernel(agg_ref, xroot_ref, g_ref, b_ref, out_ref):
    s = agg_ref[0] + agg_ref[1] + xroot_ref[...]
    m = jnp.sum(s, axis=0, keepdims=True) / N
    v = jnp.sum(s * s, axis=0, keepdims=True) / N - m * m
    y = (s - m) * jax.lax.rsqrt(v + 1e-5) * g_ref[...] + b_ref[...]
    out_ref[...] = jnp.maximum(y, 0.0)


def _tc_combine(agg2, xroot, g, b):
    return pl.pallas_call(
        _tc_combine_kernel,
        in_specs=[
            pl.BlockSpec((NC, N, H), lambda: (0, 0, 0)),
            pl.BlockSpec((N, H), lambda: (0, 0)),
            pl.BlockSpec((1, H), lambda: (0, 0)),
            pl.BlockSpec((1, H), lambda: (0, 0)),
        ],
        out_specs=pl.BlockSpec((N, H), lambda: (0, 0)),
        out_shape=jax.ShapeDtypeStruct((N, H), jnp.float32),
    )(agg2, xroot, g.reshape(1, -1), b.reshape(1, -1))


def kernel(edge_index, edge_type, emb, proj_W, proj_b,
           basis0, comp0, root0, bias0, g0, b0,
           basis1, comp1, root1, bias1, g1, b1,
           basis2, comp2, root2, bias2, g2, b2):
    src = edge_index[0].astype(jnp.int32)
    dst = edge_index[1].astype(jnp.int32)
    et = edge_type.astype(jnp.int32)

    pad = EPAD - E
    srel = jnp.concatenate([src * R + et, jnp.zeros((pad,), jnp.int32)])
    seg = jnp.concatenate([dst * R + et, jnp.full((pad,), NR, jnp.int32)])
    dstp = jnp.concatenate([dst, jnp.zeros((pad,), jnp.int32)])
    srel = srel.reshape(ROWS, CHUNK)
    seg = seg.reshape(ROWS, CHUNK)
    dstp = dstp.reshape(ROWS, CHUNK)

    w = _sc_counts_weights(seg)

    x = _tc_proj(emb, proj_W, proj_b)
    for basis, comp, root, g, b in (
            (basis0, comp0, root0, g0, b0),
            (basis1, comp1, root1, g1, b1),
            (basis2, comp2, root2, g2, b2)):
        xr, xroot = _tc_rel_matmul(x, basis, comp, root)
        agg2 = _sc_scatter(srel, dstp, w.reshape(EPAD), xr.reshape(NR, H))
        x = _tc_combine(agg2, xroot, g, b)
    return x


# speedup vs baseline: 11.5942x; 1.0074x over previous
import functools

import jax
import jax.numpy as jnp
from jax import lax
from jax.experimental import pallas as pl
from jax.experimental.pallas import tpu as pltpu
from jax.experimental.pallas import tpu_sc as plsc

N = 10000
E = 320000
R = 8
NB = 2
P = 768
H = 128

NC = 2
NS = 16
NW = NC * NS
CHUNK = 128
EPAD = 327680
ROWS = EPAD // CHUNK
TROWS = ROWS // NW
CROWS = ROWS // NS
NR = N * R
NRP = 81920
FR = 624
ZR = 48
TBAT = 16

def _mesh():
    return plsc.VectorSubcoreMesh(core_axis_name="c", subcore_axis_name="s",
                                  num_cores=NC, num_subcores=NS)


def _zero16():
    return jnp.zeros((16,), jnp.float32)


@functools.cache
def _build_counts_weights():
  return functools.partial(
    pl.kernel,
    out_type=jax.ShapeDtypeStruct((ROWS, CHUNK), jnp.float32),
    mesh=_mesh(),
    scratch_types=[
        pltpu.VMEM((CROWS, CHUNK), jnp.int32),
        pltpu.VMEM((TROWS, CHUNK), jnp.int32),
        pltpu.VMEM((TROWS, CHUNK), jnp.float32),
        pltpu.VMEM((CHUNK,), jnp.float32),
        pltpu.VMEM((CHUNK,), jnp.float32),
        pltpu.VMEM_SHARED((NRP,), jnp.float32),
        pltpu.SemaphoreType.DMA,
        pltpu.SemaphoreType.DMA,
    ],
  )(_counts_weights_body)


def _sc_counts_weights(seg):
    return _build_counts_weights()(seg)


def _counts_weights_body(seg_hbm, w_hbm, segc_v, segw_v, cw_v, ones_v, zeros_v,
                         cnt_sh, sem, sem2):
    cid = lax.axis_index("c")
    sid = lax.axis_index("s")
    wid = sid * NC + cid

    for k in range(CHUNK // 16):
        ones_v[pl.ds(k * 16, 16)] = jnp.full((16,), 1.0, jnp.float32)
        zeros_v[pl.ds(k * 16, 16)] = _zero16()
    for t in range(NRP // NS // CHUNK):
        pltpu.sync_copy(zeros_v, cnt_sh.at[pl.ds(sid * (NRP // NS) + t * CHUNK, CHUNK)])
    plsc.subcore_barrier()

    pltpu.sync_copy(seg_hbm.at[pl.ds(sid * CROWS, CROWS)], segc_v)
    copies = []
    for t in range(CROWS):
        copies.append(pltpu.make_async_copy(ones_v, cnt_sh.at[segc_v.at[t]], sem))
        copies[-1].start(add=True)
    for c in copies:
        c.wait()
    plsc.subcore_barrier()

    pltpu.sync_copy(seg_hbm.at[pl.ds(wid * TROWS, TROWS)], segw_v)
    gathers = []
    for t in range(TROWS):
        gathers.append(pltpu.make_async_copy(cnt_sh.at[segw_v.at[t]],
                                             cw_v.at[t], sem2))
        gathers[-1].start()
    for g in gathers:
        g.wait()

    base = wid * TROWS * CHUNK

    def body(i, _):
        j = i // (CHUNK // 16)
        k = i % (CHUNK // 16)
        c = cw_v[j, pl.ds(k * 16, 16)]
        w = 1.0 / jnp.maximum(c, 1.0)
        gidx = base + i * 16 + lax.broadcasted_iota(jnp.int32, (16,), 0)
        cw_v[j, pl.ds(k * 16, 16)] = jnp.where(gidx < E, w, 0.0)
        return 0

    lax.fori_loop(0, TROWS * (CHUNK // 16), body, 0)
    pltpu.sync_copy(cw_v, w_hbm.at[pl.ds(wid * TROWS, TROWS)])


@functools.cache
def _build_scatter():
  return functools.partial(
    pl.kernel,
    out_type=jax.ShapeDtypeStruct((NC, N, H), jnp.float32),
    mesh=_mesh(),
    scratch_types=[
        pltpu.VMEM((TBAT, CHUNK), jnp.int32),
        pltpu.VMEM((TBAT, CHUNK), jnp.int32),
        pltpu.VMEM((TBAT * CHUNK,), jnp.float32),
        pltpu.VMEM((CHUNK, H), jnp.float32),
        pltpu.VMEM((CHUNK, H), jnp.float32),
        pltpu.VMEM((ZR, H), jnp.float32),
        pltpu.VMEM_SHARED((N, H), jnp.float32),
        pltpu.SemaphoreType.DMA,
        pltpu.SemaphoreType.DMA,
    ],
  )(_scatter_body)


def _sc_scatter(srel, dstp, w, xr):
    return _build_scatter()(srel, dstp, w, xr)


def _scatter_body(srel_hbm, dst_hbm, w_hbm, xr_hbm, out_hbm,
                  srel_v, dst_v, w_v, rows0_v, rows1_v, zero_v, agg_sh,
                  semg, sems):
    cid = lax.axis_index("c")
    sid = lax.axis_index("s")
    wid = sid * NC + cid

    def zbody(i, _):
        j = i // (H // 16)
        k = i % (H // 16)
        zero_v[j, pl.ds(k * 16, 16)] = _zero16()
        return 0

    lax.fori_loop(0, ZR * (H // 16), zbody, 0)
    for t in range(FR // ZR):
        pltpu.sync_copy(zero_v, agg_sh.at[pl.ds(sid * FR + t * ZR, ZR)])

    @pl.when(sid == NS - 1)
    def _():
        pltpu.sync_copy(zero_v.at[pl.ds(0, N - FR * NS)],
                        agg_sh.at[pl.ds(FR * NS, N - FR * NS)])

    plsc.subcore_barrier()

    def scale(rows_v, t):
        @functools.partial(plsc.parallel_loop, 0, CHUNK // 16, unroll=2)
        def sbody(g):
            w16 = w_v[pl.ds(t * CHUNK + g * 16, 16)]
            for j in range(16):
                wb = jnp.full((16,), w16[j], jnp.float32)
                e = g * 16 + j
                for k in range(H // 16):
                    rows_v[e, pl.ds(k * 16, 16)] = (
                        rows_v[e, pl.ds(k * 16, 16)] * wb)

    for bt in range(TROWS // TBAT):
        base = wid * TROWS + bt * TBAT
        pltpu.sync_copy(srel_hbm.at[pl.ds(base, TBAT)], srel_v)
        pltpu.sync_copy(dst_hbm.at[pl.ds(base, TBAT)], dst_v)
        pltpu.sync_copy(w_hbm.at[pl.ds(base * CHUNK, TBAT * CHUNK)], w_v)

        pltpu.async_copy(xr_hbm.at[srel_v.at[0]], rows0_v, semg)

        def pbody(i, _):
            t0 = 2 * i
            t1 = t0 + 1
            pltpu.make_async_copy(xr_hbm.at[srel_v.at[t0]], rows0_v,
                                  semg).wait()

            @pl.when(i > 0)
            def _():
                pltpu.make_async_copy(rows1_v, agg_sh.at[dst_v.at[t0 - 1]],
                                      sems).wait()

            pltpu.async_copy(xr_hbm.at[srel_v.at[t1]], rows1_v, semg)
            scale(rows0_v, t0)
            pltpu.async_copy(rows0_v, agg_sh.at[dst_v.at[t0]], sems, add=True)

            pltpu.make_async_copy(xr_hbm.at[srel_v.at[t1]], rows1_v,
                                  semg).wait()
            pltpu.make_async_copy(rows0_v, agg_sh.at[dst_v.at[t0]],
                                  sems).wait()

            @pl.when(i < TBAT // 2 - 1)
            def _():
                pltpu.async_copy(xr_hbm.at[srel_v.at[t0 + 2]], rows0_v, semg)

            scale(rows1_v, t1)
            pltpu.async_copy(rows1_v, agg_sh.at[dst_v.at[t1]], sems, add=True)
            return 0

        lax.fori_loop(0, TBAT // 2, pbody, 0)
        pltpu.make_async_copy(rows1_v, agg_sh.at[dst_v.at[TBAT - 1]],
                              sems).wait()

    plsc.subcore_barrier()
    for t in range(FR // ZR):
        pltpu.sync_copy(agg_sh.at[pl.ds(sid * FR + t * ZR, ZR)], zero_v)
        pltpu.sync_copy(zero_v, out_hbm.at[cid, pl.ds(sid * FR + t * ZR, ZR)])

    @pl.when(sid == NS - 1)
    def _():
        pltpu.sync_copy(agg_sh.at[pl.ds(FR * NS, N - FR * NS)],
                        zero_v.at[pl.ds(0, N - FR * NS)])
        pltpu.sync_copy(zero_v.at[pl.ds(0, N - FR * NS)],
                        out_hbm.at[cid, pl.ds(FR * NS, N - FR * NS)])


def _tc_proj_kernel(emb_ref, w_ref, b_ref, out_ref):
    x = emb_ref[...]
    nrm = jnp.sqrt(jnp.sum(x * x, axis=1, keepdims=True))
    x = x / jnp.maximum(nrm, 1e-12)
    out_ref[...] = jnp.dot(x, w_ref[...],
                           preferred_element_type=jnp.float32) + b_ref[...]


def _tc_proj(emb, proj_W, proj_b):
    blk = 2000
    return pl.pallas_call(
        _tc_proj_kernel,
        grid=(N // blk,),
        in_specs=[
            pl.BlockSpec((blk, emb.shape[1]), lambda i: (i, 0)),
            pl.BlockSpec(proj_W.shape, lambda i: (0, 0)),
            pl.BlockSpec((1, proj_W.shape[1]), lambda i: (0, 0)),
        ],
        out_specs=pl.BlockSpec((blk, proj_W.shape[1]), lambda i: (i, 0)),
        out_shape=jax.ShapeDtypeStruct((N, proj_W.shape[1]), jnp.float32),
    )(emb, proj_W, proj_b.reshape(1, -1))


def _tc_rel_matmul_kernel(x_ref, basis_ref, comp_ref, root_ref,
                          xr_ref, xroot_ref, wcat_ref):
    @pl.when(pl.program_id(0) == 0)
    def _():
        b0 = basis_ref[0]
        b1 = basis_ref[1]
        for r in range(R):
            wcat_ref[:, r * H:(r + 1) * H] = comp_ref[r, 0] * b0 + comp_ref[r, 1] * b1
        wcat_ref[:, R * H:] = root_ref[...]

    y = jnp.dot(x_ref[...], wcat_ref[...], preferred_element_type=jnp.float32)
    xr_ref[...] = y[:, :R * H]
    xroot_ref[...] = y[:, R * H:]


def _tc_rel_matmul(x, basis, comp, root):
    din = x.shape[1]
    blk = 2000
    comp_p = jnp.zeros((R, 128), jnp.float32).at[:, :NB].set(comp)
    return pl.pallas_call(
        _tc_rel_matmul_kernel,
        grid=(N // blk,),
        in_specs=[
            pl.BlockSpec((blk, din), lambda i: (i, 0)),
            pl.BlockSpec((NB, din, H), lambda i: (0, 0, 0)),
            pl.BlockSpec((R, 128), lambda i: (0, 0)),
            pl.BlockSpec((din, H), lambda i: (0, 0)),
        ],
        out_specs=[
            pl.BlockSpec((blk, R * H), lambda i: (i, 0)),
            pl.BlockSpec((blk, H), lambda i: (i, 0)),
        ],
        out_shape=[
            jax.ShapeDtypeStruct((N, R * H), jnp.float32),
            jax.ShapeDtypeStruct((N, H), jnp.float32),
        ],
        scratch_shapes=[pltpu.VMEM((din, R * H + H), jnp.float32)],
    )(x, basis, comp_p, root)


def _tc_combine_kernel(agg_ref, xroot_ref, g_ref, b_ref, out_ref):
    s = agg_ref[0] + agg_ref[1] + xroot_ref[...]
    m = jnp.sum(s, axis=0, keepdims=True) / N
    v = jnp.sum(s * s, axis=0, keepdims=True) / N - m * m
    y = (s - m) * jax.lax.rsqrt(v + 1e-5) * g_ref[...] + b_ref[...]
    out_ref[...] = jnp.maximum(y, 0.0)


def _tc_combine(agg2, xroot, g, b):
    return pl.pallas_call(
        _tc_combine_kernel,
        in_specs=[
            pl.BlockSpec((NC, N, H), lambda: (0, 0, 0)),
            pl.BlockSpec((N, H), lambda: (0, 0)),
            pl.BlockSpec((1, H), lambda: (0, 0)),
            pl.BlockSpec((1, H), lambda: (0, 0)),
        ],
        out_specs=pl.BlockSpec((N, H), lambda: (0, 0)),
        out_shape=jax.ShapeDtypeStruct((N, H), jnp.float32),
    )(agg2, xroot, g.reshape(1, -1), b.reshape(1, -1))


def kernel(edge_index, edge_type, emb, proj_W, proj_b,
           basis0, comp0, root0, bias0, g0, b0,
           basis1, comp1, root1, bias1, g1, b1,
           basis2, comp2, root2, bias2, g2, b2):
    src = edge_index[0].astype(jnp.int32)
    dst = edge_index[1].astype(jnp.int32)
    et = edge_type.astype(jnp.int32)

    pad = EPAD - E
    srel = jnp.concatenate([src * R + et, jnp.zeros((pad,), jnp.int32)])
    seg = jnp.concatenate([dst * R + et, jnp.full((pad,), NR, jnp.int32)])
    dstp = jnp.concatenate([dst, jnp.zeros((pad,), jnp.int32)])
    srel = srel.reshape(ROWS, CHUNK)
    seg = seg.reshape(ROWS, CHUNK)
    dstp = dstp.reshape(ROWS, CHUNK)

    w = _sc_counts_weights(seg)

    x = _tc_proj(emb, proj_W, proj_b)
    for basis, comp, root, g, b in (
            (basis0, comp0, root0, g0, b0),
            (basis1, comp1, root1, g1, b1),
            (basis2, comp2, root2, g2, b2)):
        xr, xroot = _tc_rel_matmul(x, basis, comp, root)
        agg2 = _sc_scatter(srel, dstp, w.reshape(EPAD), xr.reshape(NR, H))
        x = _tc_combine(agg2, xroot, g, b)
    return x


# speedup vs baseline: 12.4114x; 1.0705x over previous
import functools

import jax
import jax.numpy as jnp
from jax import lax
from jax.experimental import pallas as pl
from jax.experimental.pallas import tpu as pltpu
from jax.experimental.pallas import tpu_sc as plsc

N = 10000
E = 320000
R = 8
NB = 2
P = 768
H = 128

NC = 2
NS = 16
NW = NC * NS
CHUNK = 128
EPAD = 327680
ROWS = EPAD // CHUNK
TROWS = ROWS // NW
CROWS = ROWS // NS
NR = N * R
NRP = 81920
FR = 624
ZR = 48
TBAT = 16
CH2 = 64
CPT = EPAD // NW // CH2
CBAT = 32
NB2 = CPT // CBAT

def _mesh():
    return plsc.VectorSubcoreMesh(core_axis_name="c", subcore_axis_name="s",
                                  num_cores=NC, num_subcores=NS)


def _zero16():
    return jnp.zeros((16,), jnp.float32)


@functools.cache
def _build_counts_weights():
  return functools.partial(
    pl.kernel,
    out_type=jax.ShapeDtypeStruct((ROWS, CHUNK), jnp.float32),
    mesh=_mesh(),
    scratch_types=[
        pltpu.VMEM((CROWS, CHUNK), jnp.int32),
        pltpu.VMEM((TROWS, CHUNK), jnp.int32),
        pltpu.VMEM((TROWS, CHUNK), jnp.float32),
        pltpu.VMEM((CHUNK,), jnp.float32),
        pltpu.VMEM((CHUNK,), jnp.float32),
        pltpu.VMEM_SHARED((NRP,), jnp.float32),
        pltpu.SemaphoreType.DMA,
        pltpu.SemaphoreType.DMA,
    ],
  )(_counts_weights_body)


def _sc_counts_weights(seg):
    return _build_counts_weights()(seg)


def _counts_weights_body(seg_hbm, w_hbm, segc_v, segw_v, cw_v, ones_v, zeros_v,
                         cnt_sh, sem, sem2):
    cid = lax.axis_index("c")
    sid = lax.axis_index("s")
    wid = sid * NC + cid

    for k in range(CHUNK // 16):
        ones_v[pl.ds(k * 16, 16)] = jnp.full((16,), 1.0, jnp.float32)
        zeros_v[pl.ds(k * 16, 16)] = _zero16()
    for t in range(NRP // NS // CHUNK):
        pltpu.sync_copy(zeros_v, cnt_sh.at[pl.ds(sid * (NRP // NS) + t * CHUNK, CHUNK)])
    plsc.subcore_barrier()

    pltpu.sync_copy(seg_hbm.at[pl.ds(sid * CROWS, CROWS)], segc_v)
    copies = []
    for t in range(CROWS):
        copies.append(pltpu.make_async_copy(ones_v, cnt_sh.at[segc_v.at[t]], sem))
        copies[-1].start(add=True)
    for c in copies:
        c.wait()
    plsc.subcore_barrier()

    pltpu.sync_copy(seg_hbm.at[pl.ds(wid * TROWS, TROWS)], segw_v)
    gathers = []
    for t in range(TROWS):
        gathers.append(pltpu.make_async_copy(cnt_sh.at[segw_v.at[t]],
                                             cw_v.at[t], sem2))
        gathers[-1].start()
    for g in gathers:
        g.wait()

    base = wid * TROWS * CHUNK

    def body(i, _):
        j = i // (CHUNK // 16)
        k = i % (CHUNK // 16)
        c = cw_v[j, pl.ds(k * 16, 16)]
        w = 1.0 / jnp.maximum(c, 1.0)
        gidx = base + i * 16 + lax.broadcasted_iota(jnp.int32, (16,), 0)
        cw_v[j, pl.ds(k * 16, 16)] = jnp.where(gidx < E, w, 0.0)
        return 0

    lax.fori_loop(0, TROWS * (CHUNK // 16), body, 0)
    pltpu.sync_copy(cw_v, w_hbm.at[pl.ds(wid * TROWS, TROWS)])


@functools.cache
def _build_scatter():
  return functools.partial(
    pl.kernel,
    out_type=jax.ShapeDtypeStruct((NC, N, H), jnp.float32),
    mesh=_mesh(),
    scratch_types=[
        pltpu.VMEM((CBAT, CH2), jnp.int32),
        pltpu.VMEM((CBAT, CH2), jnp.int32),
        pltpu.VMEM((CBAT * CH2,), jnp.float32),
        pltpu.VMEM((CH2, H), jnp.float32),
        pltpu.VMEM((CH2, H), jnp.float32),
        pltpu.VMEM((CH2, H), jnp.float32),
        pltpu.VMEM((CH2, H), jnp.float32),
        pltpu.VMEM((ZR, H), jnp.float32),
        pltpu.VMEM_SHARED((N, H), jnp.float32),
        pltpu.SemaphoreType.DMA,
        pltpu.SemaphoreType.DMA,
    ],
  )(_scatter_body)


def _sc_scatter(srel, dstp, w, xr):
    return _build_scatter()(srel, dstp, w, xr)


def _scatter_body(srel_hbm, dst_hbm, w_hbm, xr_hbm, out_hbm,
                  srel_v, dst_v, w_v, rb0, rb1, rb2, rb3, zero_v, agg_sh,
                  semg, sems):
    cid = lax.axis_index("c")
    sid = lax.axis_index("s")
    wid = sid * NC + cid

    def zbody(i, _):
        j = i // (H // 16)
        k = i % (H // 16)
        zero_v[j, pl.ds(k * 16, 16)] = _zero16()
        return 0

    lax.fori_loop(0, ZR * (H // 16), zbody, 0)
    for t in range(FR // ZR):
        pltpu.sync_copy(zero_v, agg_sh.at[pl.ds(sid * FR + t * ZR, ZR)])

    @pl.when(sid == NS - 1)
    def _():
        pltpu.sync_copy(zero_v.at[pl.ds(0, N - FR * NS)],
                        agg_sh.at[pl.ds(FR * NS, N - FR * NS)])

    plsc.subcore_barrier()

    rbufs = (rb0, rb1, rb2, rb3)

    def scale(rows_v, c):
        def sbody(g, _):
            w16 = w_v[pl.ds(c * CH2 + g * 16, 16)]
            for j in range(16):
                wb = jnp.full((16,), w16[j], jnp.float32)
                e = g * 16 + j
                for k in range(H // 16):
                    rows_v[e, pl.ds(k * 16, 16)] = (
                        rows_v[e, pl.ds(k * 16, 16)] * wb)
            return 0

        lax.fori_loop(0, CH2 // 16, sbody, 0)

    def bbody(bt, _):
        base = pl.multiple_of(wid * CPT + bt * CBAT, CBAT)
        pltpu.sync_copy(srel_hbm.at[pl.ds(base, CBAT)], srel_v)
        pltpu.sync_copy(dst_hbm.at[pl.ds(base, CBAT)], dst_v)
        pltpu.sync_copy(w_hbm.at[pl.ds(base * CH2, CBAT * CH2)], w_v)

        pltpu.async_copy(xr_hbm.at[srel_v.at[0]], rb0, semg)
        pltpu.async_copy(xr_hbm.at[srel_v.at[1]], rb1, semg)

        def rbody(k, _):
            for p in range(4):
                c = 4 * k + p
                buf = rbufs[p]
                nbuf = rbufs[(p + 2) % 4]
                if p < 2:
                    @pl.when(k > 0)
                    def _():
                        pltpu.make_async_copy(
                            nbuf, agg_sh.at[dst_v.at[c - 2]], sems).wait()

                    pltpu.async_copy(xr_hbm.at[srel_v.at[c + 2]], nbuf, semg)
                else:
                    pltpu.make_async_copy(
                        nbuf, agg_sh.at[dst_v.at[c - 2]], sems).wait()

                    @pl.when(k < CBAT // 4 - 1)
                    def _():
                        pltpu.async_copy(
                            xr_hbm.at[srel_v.at[c + 2]], nbuf, semg)

                pltpu.make_async_copy(xr_hbm.at[srel_v.at[c]], buf,
                                      semg).wait()
                scale(buf, c)
                pltpu.async_copy(buf, agg_sh.at[dst_v.at[c]], sems, add=True)
            return 0

        lax.fori_loop(0, CBAT // 4, rbody, 0)
        pltpu.make_async_copy(rb2, agg_sh.at[dst_v.at[CBAT - 2]], sems).wait()
        pltpu.make_async_copy(rb3, agg_sh.at[dst_v.at[CBAT - 1]], sems).wait()
        return 0

    lax.fori_loop(0, NB2, bbody, 0)

    plsc.subcore_barrier()
    for t in range(FR // ZR):
        pltpu.sync_copy(agg_sh.at[pl.ds(sid * FR + t * ZR, ZR)], zero_v)
        pltpu.sync_copy(zero_v, out_hbm.at[cid, pl.ds(sid * FR + t * ZR, ZR)])

    @pl.when(sid == NS - 1)
    def _():
        pltpu.sync_copy(agg_sh.at[pl.ds(FR * NS, N - FR * NS)],
                        zero_v.at[pl.ds(0, N - FR * NS)])
        pltpu.sync_copy(zero_v.at[pl.ds(0, N - FR * NS)],
                        out_hbm.at[cid, pl.ds(FR * NS, N - FR * NS)])


def _tc_proj_kernel(emb_ref, w_ref, b_ref, out_ref):
    x = emb_ref[...]
    nrm = jnp.sqrt(jnp.sum(x * x, axis=1, keepdims=True))
    x = x / jnp.maximum(nrm, 1e-12)
    out_ref[...] = jnp.dot(x, w_ref[...],
                           preferred_element_type=jnp.float32) + b_ref[...]


def _tc_proj(emb, proj_W, proj_b):
    blk = 2000
    return pl.pallas_call(
        _tc_proj_kernel,
        grid=(N // blk,),
        in_specs=[
            pl.BlockSpec((blk, emb.shape[1]), lambda i: (i, 0)),
            pl.BlockSpec(proj_W.shape, lambda i: (0, 0)),
            pl.BlockSpec((1, proj_W.shape[1]), lambda i: (0, 0)),
        ],
        out_specs=pl.BlockSpec((blk, proj_W.shape[1]), lambda i: (i, 0)),
        out_shape=jax.ShapeDtypeStruct((N, proj_W.shape[1]), jnp.float32),
    )(emb, proj_W, proj_b.reshape(1, -1))


def _tc_rel_matmul_kernel(x_ref, basis_ref, comp_ref, root_ref,
                          xr_ref, xroot_ref, wcat_ref):
    @pl.when(pl.program_id(0) == 0)
    def _():
        b0 = basis_ref[0]
        b1 = basis_ref[1]
        for r in range(R):
            wcat_ref[:, r * H:(r + 1) * H] = comp_ref[r, 0] * b0 + comp_ref[r, 1] * b1
        wcat_ref[:, R * H:] = root_ref[...]

    y = jnp.dot(x_ref[...], wcat_ref[...], preferred_element_type=jnp.float32)
    xr_ref[...] = y[:, :R * H]
    xroot_ref[...] = y[:, R * H:]


def _tc_rel_matmul(x, basis, comp, root):
    din = x.shape[1]
    blk = 2000
    comp_p = jnp.zeros((R, 128), jnp.float32).at[:, :NB].set(comp)
    return pl.pallas_call(
        _tc_rel_matmul_kernel,
        grid=(N // blk,),
        in_specs=[
            pl.BlockSpec((blk, din), lambda i: (i, 0)),
            pl.BlockSpec((NB, din, H), lambda i: (0, 0, 0)),
            pl.BlockSpec((R, 128), lambda i: (0, 0)),
            pl.BlockSpec((din, H), lambda i: (0, 0)),
        ],
        out_specs=[
            pl.BlockSpec((blk, R * H), lambda i: (i, 0)),
            pl.BlockSpec((blk, H), lambda i: (i, 0)),
        ],
        out_shape=[
            jax.ShapeDtypeStruct((N, R * H), jnp.float32),
            jax.ShapeDtypeStruct((N, H), jnp.float32),
        ],
        scratch_shapes=[pltpu.VMEM((din, R * H + H), jnp.float32)],
    )(x, basis, comp_p, root)


def _tc_combine_kernel(agg_ref, xroot_ref, g_ref, b_ref, out_ref):
    s = agg_ref[0] + agg_ref[1] + xroot_ref[...]
    m = jnp.sum(s, axis=0, keepdims=True) / N
    v = jnp.sum(s * s, axis=0, keepdims=True) / N - m * m
    y = (s - m) * jax.lax.rsqrt(v + 1e-5) * g_ref[...] + b_ref[...]
    out_ref[...] = jnp.maximum(y, 0.0)


def _tc_combine(agg2, xroot, g, b):
    return pl.pallas_call(
        _tc_combine_kernel,
        in_specs=[
            pl.BlockSpec((NC, N, H), lambda: (0, 0, 0)),
            pl.BlockSpec((N, H), lambda: (0, 0)),
            pl.BlockSpec((1, H), lambda: (0, 0)),
            pl.BlockSpec((1, H), lambda: (0, 0)),
        ],
        out_specs=pl.BlockSpec((N, H), lambda: (0, 0)),
        out_shape=jax.ShapeDtypeStruct((N, H), jnp.float32),
    )(agg2, xroot, g.reshape(1, -1), b.reshape(1, -1))


def kernel(edge_index, edge_type, emb, proj_W, proj_b,
           basis0, comp0, root0, bias0, g0, b0,
           basis1, comp1, root1, bias1, g1, b1,
           basis2, comp2, root2, bias2, g2, b2):
    src = edge_index[0].astype(jnp.int32)
    dst = edge_index[1].astype(jnp.int32)
    et = edge_type.astype(jnp.int32)

    pad = EPAD - E
    srel = jnp.concatenate([src * R + et, jnp.zeros((pad,), jnp.int32)])
    seg = jnp.concatenate([dst * R + et, jnp.full((pad,), NR, jnp.int32)])
    dstp = jnp.concatenate([dst, jnp.zeros((pad,), jnp.int32)])
    srel = srel.reshape(EPAD // CH2, CH2)
    seg = seg.reshape(ROWS, CHUNK)
    dstp = dstp.reshape(EPAD // CH2, CH2)

    w = _sc_counts_weights(seg)

    x = _tc_proj(emb, proj_W, proj_b)
    for basis, comp, root, g, b in (
            (basis0, comp0, root0, g0, b0),
            (basis1, comp1, root1, g1, b1),
            (basis2, comp2, root2, g2, b2)):
        xr, xroot = _tc_rel_matmul(x, basis, comp, root)
        agg2 = _sc_scatter(srel, dstp, w.reshape(EPAD), xr.reshape(NR, H))
        x = _tc_combine(agg2, xroot, g, b)
    return x


# speedup vs baseline: 12.4202x; 1.0007x over previous
import functools

import jax
import jax.numpy as jnp
import numpy as np
from jax import lax
from jax.experimental import pallas as pl
from jax.experimental.pallas import tpu as pltpu
from jax.experimental.pallas import tpu_sc as plsc

N = 10000
E = 320000
R = 8
NB = 2
P = 768
H = 128

NC = 2
NS = 16
NW = NC * NS
CHUNK = 128
EPAD = 327680
ROWS = EPAD // CHUNK
TROWS = ROWS // NW
CROWS = ROWS // NS
NR = N * R
NRP = 81920
FR = 624
ZR = 48
TBAT = 16
CH2 = 64
CPT = EPAD // NW // CH2
CBAT = 32
NB2 = CPT // CBAT

def _mesh():
    return plsc.VectorSubcoreMesh(core_axis_name="c", subcore_axis_name="s",
                                  num_cores=NC, num_subcores=NS)


def _zero16():
    return jnp.zeros((16,), jnp.float32)


@functools.cache
def _build_counts_weights():
  return functools.partial(
    pl.kernel,
    out_type=jax.ShapeDtypeStruct((ROWS, CHUNK), jnp.float32),
    mesh=_mesh(),
    scratch_types=[
        pltpu.VMEM((CROWS, CHUNK), jnp.int32),
        pltpu.VMEM((TROWS, CHUNK), jnp.int32),
        pltpu.VMEM((TROWS, CHUNK), jnp.float32),
        pltpu.VMEM((CHUNK,), jnp.float32),
        pltpu.VMEM((CHUNK,), jnp.float32),
        pltpu.VMEM_SHARED((NRP,), jnp.float32),
        pltpu.SemaphoreType.DMA,
        pltpu.SemaphoreType.DMA,
    ],
  )(_counts_weights_body)


def _sc_counts_weights(seg):
    return _build_counts_weights()(seg)


def _counts_weights_body(seg_hbm, w_hbm, segc_v, segw_v, cw_v, ones_v, zeros_v,
                         cnt_sh, sem, sem2):
    cid = lax.axis_index("c")
    sid = lax.axis_index("s")
    wid = sid * NC + cid

    for k in range(CHUNK // 16):
        ones_v[pl.ds(k * 16, 16)] = jnp.full((16,), 1.0, jnp.float32)
        zeros_v[pl.ds(k * 16, 16)] = _zero16()
    for t in range(NRP // NS // CHUNK):
        pltpu.sync_copy(zeros_v, cnt_sh.at[pl.ds(sid * (NRP // NS) + t * CHUNK, CHUNK)])
    plsc.subcore_barrier()

    pltpu.sync_copy(seg_hbm.at[pl.ds(sid * CROWS, CROWS)], segc_v)
    copies = []
    for t in range(CROWS):
        copies.append(pltpu.make_async_copy(ones_v, cnt_sh.at[segc_v.at[t]], sem))
        copies[-1].start(add=True)
    for c in copies:
        c.wait()
    plsc.subcore_barrier()

    pltpu.sync_copy(seg_hbm.at[pl.ds(wid * TROWS, TROWS)], segw_v)
    gathers = []
    for t in range(TROWS):
        gathers.append(pltpu.make_async_copy(cnt_sh.at[segw_v.at[t]],
                                             cw_v.at[t], sem2))
        gathers[-1].start()
    for g in gathers:
        g.wait()

    base = wid * TROWS * CHUNK

    def body(i, _):
        j = i // (CHUNK // 16)
        k = i % (CHUNK // 16)
        c = cw_v[j, pl.ds(k * 16, 16)]
        w = 1.0 / jnp.maximum(c, 1.0)
        gidx = base + i * 16 + lax.broadcasted_iota(jnp.int32, (16,), 0)
        cw_v[j, pl.ds(k * 16, 16)] = jnp.where(gidx < E, w, 0.0)
        return 0

    lax.fori_loop(0, TROWS * (CHUNK // 16), body, 0)
    pltpu.sync_copy(cw_v, w_hbm.at[pl.ds(wid * TROWS, TROWS)])


@functools.cache
def _build_scatter():
  return functools.partial(
    pl.kernel,
    out_type=jax.ShapeDtypeStruct((NC, N, H), jnp.float32),
    mesh=_mesh(),
    scratch_types=[
        pltpu.VMEM((CBAT, CH2), jnp.int32),
        pltpu.VMEM((CBAT, CH2), jnp.int32),
        pltpu.VMEM((CBAT * CH2,), jnp.float32),
        pltpu.VMEM((CH2, H), jnp.float32),
        pltpu.VMEM((CH2, H), jnp.float32),
        pltpu.VMEM((CH2, H), jnp.float32),
        pltpu.VMEM((CH2, H), jnp.float32),
        pltpu.VMEM((ZR, H), jnp.float32),
        pltpu.VMEM_SHARED((N, H), jnp.float32),
        pltpu.SemaphoreType.DMA,
        pltpu.SemaphoreType.DMA,
    ],
  )(_scatter_body)


def _sc_scatter(srel, dstp, w, xr):
    return _build_scatter()(srel, dstp, w, xr)


def _scatter_body(srel_hbm, dst_hbm, w_hbm, xr_hbm, out_hbm,
                  srel_v, dst_v, w_v, rb0, rb1, rb2, rb3,
                  zero_v, agg_sh, semg, sems):
    cid = lax.axis_index("c")
    sid = lax.axis_index("s")
    wid = sid * NC + cid

    def zbody(i, _):
        j = i // (H // 16)
        k = i % (H // 16)
        zero_v[j, pl.ds(k * 16, 16)] = _zero16()
        return 0

    lax.fori_loop(0, ZR * (H // 16), zbody, 0)
    for t in range(FR // ZR):
        pltpu.sync_copy(zero_v, agg_sh.at[pl.ds(sid * FR + t * ZR, ZR)])

    @pl.when(sid == NS - 1)
    def _():
        pltpu.sync_copy(zero_v.at[pl.ds(0, N - FR * NS)],
                        agg_sh.at[pl.ds(FR * NS, N - FR * NS)])

    plsc.subcore_barrier()

    gbufs = (rb0, rb1, rb2, rb3)

    def scale(gbuf, c):
        def sbody(g, _):
            w16 = w_v[pl.ds(c * CH2 + g * 16, 16)]
            for j in range(16):
                wb = jnp.full((16,), w16[j], jnp.float32)
                e = g * 16 + j
                for k in range(H // 16):
                    gbuf[e, pl.ds(k * 16, 16)] = (
                        gbuf[e, pl.ds(k * 16, 16)] * wb)
            return 0

        lax.fori_loop(0, CH2 // 16, sbody, 0)

    def bbody(bt, _):
        base = pl.multiple_of(wid * CPT + bt * CBAT, CBAT)
        pltpu.sync_copy(srel_hbm.at[pl.ds(base, CBAT)], srel_v)
        pltpu.sync_copy(dst_hbm.at[pl.ds(base, CBAT)], dst_v)
        pltpu.sync_copy(w_hbm.at[pl.ds(base * CH2, CBAT * CH2)], w_v)

        pltpu.async_copy(xr_hbm.at[srel_v.at[0]], rb0, semg)
        pltpu.async_copy(xr_hbm.at[srel_v.at[1]], rb1, semg)

        def rbody(k, _):
            for p in range(4):
                c = 4 * k + p
                gbuf = gbufs[p]
                ngbuf = gbufs[(p + 2) % 4]
                if p < 2:
                    @pl.when(k > 0)
                    def _():
                        pltpu.make_async_copy(
                            ngbuf, agg_sh.at[dst_v.at[c - 2]], sems).wait()

                    pltpu.async_copy(xr_hbm.at[srel_v.at[c + 2]], ngbuf, semg)
                else:
                    pltpu.make_async_copy(
                        ngbuf, agg_sh.at[dst_v.at[c - 2]], sems).wait()

                    @pl.when(k < CBAT // 4 - 1)
                    def _():
                        pltpu.async_copy(
                            xr_hbm.at[srel_v.at[c + 2]], ngbuf, semg)

                pltpu.make_async_copy(xr_hbm.at[srel_v.at[c]], gbuf,
                                      semg).wait()
                scale(gbuf, c)
                pltpu.async_copy(gbuf, agg_sh.at[dst_v.at[c]], sems, add=True)
            return 0

        lax.fori_loop(0, CBAT // 4, rbody, 0)
        pltpu.make_async_copy(rb2, agg_sh.at[dst_v.at[CBAT - 2]], sems).wait()
        pltpu.make_async_copy(rb3, agg_sh.at[dst_v.at[CBAT - 1]], sems).wait()
        return 0

    lax.fori_loop(0, NB2, bbody, 0)

    plsc.subcore_barrier()
    for t in range(FR // ZR):
        pltpu.sync_copy(agg_sh.at[pl.ds(sid * FR + t * ZR, ZR)], zero_v)
        pltpu.sync_copy(zero_v, out_hbm.at[cid, pl.ds(sid * FR + t * ZR, ZR)])

    @pl.when(sid == NS - 1)
    def _():
        pltpu.sync_copy(agg_sh.at[pl.ds(FR * NS, N - FR * NS)],
                        zero_v.at[pl.ds(0, N - FR * NS)])
        pltpu.sync_copy(zero_v.at[pl.ds(0, N - FR * NS)],
                        out_hbm.at[cid, pl.ds(FR * NS, N - FR * NS)])


def _tc_proj_kernel(emb_ref, w_ref, b_ref, out_ref):
    x = emb_ref[...]
    nrm = jnp.sqrt(jnp.sum(x * x, axis=1, keepdims=True))
    x = x / jnp.maximum(nrm, 1e-12)
    out_ref[...] = jnp.dot(x, w_ref[...],
                           preferred_element_type=jnp.float32) + b_ref[...]


def _tc_proj(emb, proj_W, proj_b):
    blk = 2000
    return pl.pallas_call(
        _tc_proj_kernel,
        grid=(N // blk,),
        in_specs=[
            pl.BlockSpec((blk, emb.shape[1]), lambda i: (i, 0)),
            pl.BlockSpec(proj_W.shape, lambda i: (0, 0)),
            pl.BlockSpec((1, proj_W.shape[1]), lambda i: (0, 0)),
        ],
        out_specs=pl.BlockSpec((blk, proj_W.shape[1]), lambda i: (i, 0)),
        out_shape=jax.ShapeDtypeStruct((N, proj_W.shape[1]), jnp.float32),
    )(emb, proj_W, proj_b.reshape(1, -1))


def _tc_rel_matmul_kernel(x_ref, basis_ref, comp_ref, root_ref,
                          xr_ref, xroot_ref, wcat_ref):
    @pl.when(pl.program_id(0) == 0)
    def _():
        b0 = basis_ref[0]
        b1 = basis_ref[1]
        for r in range(R):
            wcat_ref[:, r * H:(r + 1) * H] = comp_ref[r, 0] * b0 + comp_ref[r, 1] * b1
        wcat_ref[:, R * H:] = root_ref[...]

    y = jnp.dot(x_ref[...], wcat_ref[...], preferred_element_type=jnp.float32)
    xr_ref[...] = y[:, :R * H]
    xroot_ref[...] = y[:, R * H:]


def _tc_rel_matmul(x, basis, comp, root):
    din = x.shape[1]
    blk = 2000
    comp_p = jnp.zeros((R, 128), jnp.float32).at[:, :NB].set(comp)
    return pl.pallas_call(
        _tc_rel_matmul_kernel,
        grid=(N // blk,),
        in_specs=[
            pl.BlockSpec((blk, din), lambda i: (i, 0)),
            pl.BlockSpec((NB, din, H), lambda i: (0, 0, 0)),
            pl.BlockSpec((R, 128), lambda i: (0, 0)),
            pl.BlockSpec((din, H), lambda i: (0, 0)),
        ],
        out_specs=[
            pl.BlockSpec((blk, R * H), lambda i: (i, 0)),
            pl.BlockSpec((blk, H), lambda i: (i, 0)),
        ],
        out_shape=[
            jax.ShapeDtypeStruct((N, R * H), jnp.float32),
            jax.ShapeDtypeStruct((N, H), jnp.float32),
        ],
        scratch_shapes=[pltpu.VMEM((din, R * H + H), jnp.float32)],
    )(x, basis, comp_p, root)


def _tc_combine_kernel(agg_ref, xroot_ref, g_ref, b_ref, out_ref):
    s = agg_ref[0] + agg_ref[1] + xroot_ref[...]
    m = jnp.sum(s, axis=0, keepdims=True) / N
    v = jnp.sum(s * s, axis=0, keepdims=True) / N - m * m
    y = (s - m) * jax.lax.rsqrt(v + 1e-5) * g_ref[...] + b_ref[...]
    out_ref[...] = jnp.maximum(y, 0.0)


def _tc_combine(agg2, xroot, g, b):
    return pl.pallas_call(
        _tc_combine_kernel,
        in_specs=[
            pl.BlockSpec((NC, N, H), lambda: (0, 0, 0)),
            pl.BlockSpec((N, H), lambda: (0, 0)),
            pl.BlockSpec((1, H), lambda: (0, 0)),
            pl.BlockSpec((1, H), lambda: (0, 0)),
        ],
        out_specs=pl.BlockSpec((N, H), lambda: (0, 0)),
        out_shape=jax.ShapeDtypeStruct((N, H), jnp.float32),
    )(agg2, xroot, g.reshape(1, -1), b.reshape(1, -1))


def kernel(edge_index, edge_type, emb, proj_W, proj_b,
           basis0, comp0, root0, bias0, g0, b0,
           basis1, comp1, root1, bias1, g1, b1,
           basis2, comp2, root2, bias2, g2, b2):
    src = edge_index[0].astype(jnp.int32)
    dst = edge_index[1].astype(jnp.int32)
    et = edge_type.astype(jnp.int32)

    pad = EPAD - E
    srel = jnp.concatenate([src * R + et, jnp.zeros((pad,), jnp.int32)])
    seg = jnp.concatenate([dst * R + et, jnp.full((pad,), NR, jnp.int32)])
    dstp = jnp.concatenate([dst, jnp.zeros((pad,), jnp.int32)])
    srel = srel.reshape(EPAD // CH2, CH2)
    seg = seg.reshape(ROWS, CHUNK)
    dstp = dstp.reshape(EPAD // CH2, CH2)

    w = _sc_counts_weights(seg)

    x = _tc_proj(emb, proj_W, proj_b)
    for basis, comp, root, g, b in (
            (basis0, comp0, root0, g0, b0),
            (basis1, comp1, root1, g1, b1),
            (basis2, comp2, root2, g2, b2)):
        xr, xroot = _tc_rel_matmul(x, basis, comp, root)
        agg2 = _sc_scatter(srel, dstp, w.reshape(EPAD), xr.reshape(NR, H))
        x = _tc_combine(agg2, xroot, g, b)
    return x
